# baseline (speedup 1.0000x reference)
import functools

import jax
import jax.numpy as jnp
from jax import lax
from jax.experimental import pallas as pl
from jax.experimental.pallas import tpu as pltpu
from jax.experimental.pallas import tpu_sc as plsc

_B = 2
_N = 10000
_CIN = 128
_HID = 128
_SEQ = 12
_T1 = 10
_T2 = 8
_NSLICE = _B * _T1
_OUT = 12

_NSC = 2
_TPS = 16
_NTILES = _NSC * _TPS
_CHUNK = 128
_CPT = 80
_EPT = _CHUNK * _CPT
_E_PAD = _NTILES * _EPT
_STRIPE = 632
_ROWS = _TPS * _STRIPE

_BN = 400
_NB = _N // _BN

_NPAD = _ROWS
_HALF = 5056
_ROWS2 = 5120
_STRIPE2 = _ROWS2 // _TPS
_CPT2 = 84
_CAP = _TPS * _CPT2 * _CHUNK
_BNL = 632


def _dis_from_deg(deg_blk):
    return jnp.where(deg_blk > 0.0, lax.rsqrt(jnp.maximum(deg_blk, 1.0)), 0.0)


def _sc_degree(src_rm3):
    mesh = plsc.VectorSubcoreMesh(core_axis_name="c", subcore_axis_name="s",
                                  num_cores=_NSC, num_subcores=_TPS)

    @functools.partial(
        pl.kernel,
        out_type=jax.ShapeDtypeStruct((_NSC, _ROWS, _HID), jnp.float32),
        mesh=mesh,
        scratch_types=[
            pltpu.VMEM((_CPT, _CHUNK), jnp.int32),
            pltpu.VMEM((_CHUNK, _HID), jnp.float32),
            pltpu.VMEM_SHARED((_ROWS, _HID), jnp.float32),
        ],
    )
    def deg_kernel(src_hbm, out_hbm, idx_v, buf, acc):
        sc = lax.axis_index("c")
        ts = lax.axis_index("s")
        w = sc * _TPS + ts
        pltpu.sync_copy(src_hbm.at[w], idx_v)

        def fill(val):
            def frow(i, carry):
                def fcol(j, inner):
                    buf[i, pl.ds(j * 16, 16)] = jnp.full((16,), val,
                                                         jnp.float32)
                    return inner
                return lax.fori_loop(0, _HID // 16, fcol, carry)
            lax.fori_loop(0, _CHUNK, frow, 0)

        fill(0.0)
        for r in range(4):
            pltpu.sync_copy(buf,
                            acc.at[pl.ds(ts * _STRIPE + r * _CHUNK, _CHUNK)])
        rem = _STRIPE - 4 * _CHUNK
        pltpu.sync_copy(buf.at[pl.ds(0, rem)],
                        acc.at[pl.ds(ts * _STRIPE + 4 * _CHUNK, rem)])
        fill(1.0)
        plsc.subcore_barrier()

        def add_chunk(c, carry):
            pltpu.sync_copy(buf, acc.at[idx_v.at[c]], add=True)
            return carry

        lax.fori_loop(0, _CPT, add_chunk, 0)
        plsc.subcore_barrier()
        pltpu.sync_copy(acc.at[pl.ds(ts * _STRIPE, _STRIPE)],
                        out_hbm.at[sc, pl.ds(ts * _STRIPE, _STRIPE)])

    return deg_kernel(src_rm3)


def _sc_lap(u_flat, srcoff, dstl):
    mesh = plsc.VectorSubcoreMesh(core_axis_name="c", subcore_axis_name="s",
                                  num_cores=_NSC, num_subcores=_TPS)

    @functools.partial(
        pl.kernel,
        out_type=jax.ShapeDtypeStruct((_NSC, _NSLICE, _ROWS2, _HID),
                                      jnp.float32),
        mesh=mesh,
        scratch_types=[
            pltpu.VMEM((_CPT2, _CHUNK), jnp.int32),
            pltpu.VMEM((_CPT2, _CHUNK), jnp.int32),
            pltpu.VMEM((_CHUNK, _HID), jnp.float32),
            pltpu.VMEM((_CHUNK, _HID), jnp.float32),
            pltpu.VMEM((_CHUNK, _HID), jnp.float32),
            pltpu.VMEM((_CHUNK, _HID), jnp.float32),
            pltpu.VMEM_SHARED((_ROWS2, _HID), jnp.float32),
            pltpu.SemaphoreType.DMA,
            pltpu.SemaphoreType.DMA,
            pltpu.SemaphoreType.DMA,
            pltpu.SemaphoreType.DMA,
        ],
    )
    def lap_kernel(u_hbm, src_hbm, dst_hbm, out_hbm,
                   src_v, dst_v, g0, g1, g2, g3, acc, s0, s1, s2, s3):
        sc = lax.axis_index("c")
        ts = lax.axis_index("s")
        gbufs = (g0, g1, g2, g3)
        sems = (s0, s1, s2, s3)
        pltpu.sync_copy(dst_hbm.at[sc, ts], dst_v)

        def zero_stripe():
            def zrow(i, carry):
                def zcol(j, inner):
                    g0[i, pl.ds(j * 16, 16)] = jnp.zeros((16,), jnp.float32)
                    return inner
                return lax.fori_loop(0, _HID // 16, zcol, carry)

            lax.fori_loop(0, _CHUNK, zrow, 0)
            for r in range(2):
                pltpu.sync_copy(
                    g0, acc.at[pl.ds(ts * _STRIPE2 + r * _CHUNK, _CHUNK)])
            rem = _STRIPE2 - 2 * _CHUNK
            pltpu.sync_copy(g0.at[pl.ds(0, rem)],
                            acc.at[pl.ds(ts * _STRIPE2 + 2 * _CHUNK, rem)])

        zero_stripe()

        def slice_body(s, carry):
            pltpu.sync_copy(src_hbm.at[s, sc, ts], src_v)
            plsc.subcore_barrier()
            for b in range(4):
                pltpu.async_copy(u_hbm.at[src_v.at[b]], gbufs[b], sems[b])

            def quad(p, inner):
                for b in range(4):
                    c = 4 * p + b
                    pltpu.make_async_copy(
                        u_hbm.at[pl.ds(0, _CHUNK)], gbufs[b],
                        sems[b]).wait()
                    pltpu.sync_copy(gbufs[b], acc.at[dst_v.at[c]], add=True)

                    @pl.when(p < _CPT2 // 4 - 1)
                    def _():
                        pltpu.async_copy(u_hbm.at[src_v.at[c + 4]],
                                         gbufs[b], sems[b])
                return inner

            lax.fori_loop(0, _CPT2 // 4, quad, 0)
            plsc.subcore_barrier()
            pltpu.sync_copy(acc.at[pl.ds(ts * _STRIPE2, _STRIPE2)],
                            out_hbm.at[sc, s, pl.ds(ts * _STRIPE2, _STRIPE2)])
            zero_stripe()
            return carry

        lax.fori_loop(0, _NSLICE, slice_body, 0)

    return lap_kernel(u_flat, srcoff, dstl)


def _k1_body(x0, x1, x2, w_ref, b_ref, deg_ref, t0_ref, u0_ref):
    xs = (x0[0, 0], x1[0, 0], x2[0, 0])
    gates = []
    for g in range(3):
        acc = jnp.zeros((_BNL, _HID), jnp.float32)
        for k in range(3):
            acc = acc + jnp.dot(xs[k], w_ref[g, k],
                                preferred_element_type=jnp.float32)
        gates.append(acc + b_ref[g][None, :])
    p, q, r = gates
    h = jnp.maximum(p * jax.nn.sigmoid(q) + r, 0.0)
    dis = _dis_from_deg(deg_ref[...])
    t0_ref[0] = h
    u0_ref[0] = h * dis


def _k1(xt, w1s, b1s, deg):
    grid = (_NSLICE, _NPAD // _BNL)

    def xmap(k):
        return lambda s, nb: (s // _T1, (s % _T1) + k, nb, 0)

    return pl.pallas_call(
        _k1_body,
        grid=grid,
        in_specs=[
            pl.BlockSpec((1, 1, _BNL, _CIN), xmap(0)),
            pl.BlockSpec((1, 1, _BNL, _CIN), xmap(1)),
            pl.BlockSpec((1, 1, _BNL, _CIN), xmap(2)),
            pl.BlockSpec((3, 3, _CIN, _HID), lambda s, nb: (0, 0, 0, 0)),
            pl.BlockSpec((3, _HID), lambda s, nb: (0, 0)),
            pl.BlockSpec((_BNL, 1), lambda s, nb: (nb, 0)),
        ],
        out_specs=[
            pl.BlockSpec((1, _BNL, _HID), lambda s, nb: (s, nb, 0)),
            pl.BlockSpec((1, _BNL, _HID), lambda s, nb: (s, nb, 0)),
        ],
        out_shape=[
            jax.ShapeDtypeStruct((_NSLICE, _NPAD, _HID), jnp.float32),
            jax.ShapeDtypeStruct((_NSLICE, _NPAD, _HID), jnp.float32),
        ],
    )(xt, xt, xt, w1s, b1s, deg)


def _k2_body(a, t0, deg_ref, w0_ref, w1_ref, u1_ref, p_ref):
    dis = _dis_from_deg(deg_ref[...])
    tx1 = -(dis * a[0, 0])
    u1_ref[0] = dis * tx1
    p_ref[0] = (jnp.dot(t0[0], w0_ref[...], preferred_element_type=jnp.float32)
                + jnp.dot(tx1, w1_ref[...], preferred_element_type=jnp.float32))


def _k2(acc1, t0, deg, w0, w1):
    grid = (_NSLICE, _NSC, _HALF // _BNL)
    nmap = lambda s, h, nb: (s, h * (_HALF // _BNL) + nb, 0)
    return pl.pallas_call(
        _k2_body,
        grid=grid,
        in_specs=[
            pl.BlockSpec((1, 1, _BNL, _HID), lambda s, h, nb: (h, s, nb, 0)),
            pl.BlockSpec((1, _BNL, _HID), nmap),
            pl.BlockSpec((_BNL, 1), lambda s, h, nb: (h * (_HALF // _BNL) + nb, 0)),
            pl.BlockSpec((_HID, _HID), lambda s, h, nb: (0, 0)),
            pl.BlockSpec((_HID, _HID), lambda s, h, nb: (0, 0)),
        ],
        out_specs=[
            pl.BlockSpec((1, _BNL, _HID), nmap),
            pl.BlockSpec((1, _BNL, _HID), nmap),
        ],
        out_shape=[
            jax.ShapeDtypeStruct((_NSLICE, _NPAD, _HID), jnp.float32),
            jax.ShapeDtypeStruct((_NSLICE, _NPAD, _HID), jnp.float32),
        ],
    )(acc1, t0, deg, w0, w1)


def _k4_body(a, t0, p, deg_ref, w2_ref, cb_ref, h_ref):
    dis = _dis_from_deg(deg_ref[...])
    tx2 = -2.0 * (dis * a[0, 0]) - t0[0]
    out = (p[0]
           + jnp.dot(tx2, w2_ref[...], preferred_element_type=jnp.float32)
           + cb_ref[0][None, :])
    h_ref[0] = jnp.maximum(out, 0.0)


def _k4(acc2, t0, p, deg, w2, cb):
    grid = (_NSLICE, _NSC, _HALF // _BNL)
    nmap = lambda s, h, nb: (s, h * (_HALF // _BNL) + nb, 0)
    return pl.pallas_call(
        _k4_body,
        grid=grid,
        in_specs=[
            pl.BlockSpec((1, 1, _BNL, _HID), lambda s, h, nb: (h, s, nb, 0)),
            pl.BlockSpec((1, _BNL, _HID), nmap),
            pl.BlockSpec((1, _BNL, _HID), nmap),
            pl.BlockSpec((_BNL, 1), lambda s, h, nb: (h * (_HALF // _BNL) + nb, 0)),
            pl.BlockSpec((_HID, _HID), lambda s, h, nb: (0, 0)),
            pl.BlockSpec((1, _HID), lambda s, h, nb: (0, 0)),
        ],
        out_specs=pl.BlockSpec((1, _BNL, _HID), nmap),
        out_shape=jax.ShapeDtypeStruct((_NSLICE, _NPAD, _HID), jnp.float32),
    )(acc2, t0, p, deg, w2, cb)


def _k5_body(h_ref, w_ref, b_ref, bnw_ref, bnb_ref, o_ref):
    outs = []
    for b in range(_B):
        for t in range(_T2):
            gates = []
            for g in range(3):
                acc = jnp.zeros((_BN, _OUT), jnp.float32)
                for k in range(3):
                    acc = acc + jnp.dot(h_ref[b, t + k], w_ref[g, k],
                                        preferred_element_type=jnp.float32)
                gates.append(acc + b_ref[g][None, :])
            p, q, r = gates
            outs.append(jnp.maximum(p * jax.nn.sigmoid(q) + r, 0.0))
    stack = jnp.stack(outs, axis=0)
    cnt = float(_B * _T2 * _OUT)
    mean = jnp.sum(jnp.sum(stack, axis=0), axis=1) / cnt
    msq = jnp.sum(jnp.sum(stack * stack, axis=0), axis=1) / cnt
    var = msq - mean * mean
    scale = lax.rsqrt(var + 1e-5) * bnw_ref[:, 0]
    shift = bnb_ref[:, 0]
    last = jnp.stack([outs[_T2 - 1], outs[2 * _T2 - 1]], axis=0)
    o_ref[...] = ((last - mean[None, :, None]) * scale[None, :, None]
                  + shift[None, :, None])


def _k5(h5, w2s, b2s, bnw, bnb):
    grid = (_NB,)
    return pl.pallas_call(
        _k5_body,
        grid=grid,
        in_specs=[
            pl.BlockSpec((_B, _T1, _BN, _HID), lambda nb: (0, 0, nb, 0)),
            pl.BlockSpec((3, 3, _HID, _OUT), lambda nb: (0, 0, 0, 0)),
            pl.BlockSpec((3, _OUT), lambda nb: (0, 0)),
            pl.BlockSpec((_BN, 1), lambda nb: (nb, 0)),
            pl.BlockSpec((_BN, 1), lambda nb: (nb, 0)),
        ],
        out_specs=pl.BlockSpec((_B, _BN, _OUT), lambda nb: (0, nb, 0)),
        out_shape=jax.ShapeDtypeStruct((_B, _N, _OUT), jnp.float32),
    )(h5, w2s, b2s, bnw, bnb)


def kernel(X, edge_index, w11, b11, w12, b12, w13, b13, cheb_w, cheb_b,
           w21, b21, w22, b22, w23, b23, bn_w, bn_b):
    src = edge_index[0].astype(jnp.int32)
    dst = edge_index[1].astype(jnp.int32)
    npad = _E_PAD - src.shape[0]
    srcp = jnp.concatenate([src, jnp.zeros((npad,), jnp.int32)])
    dstp = jnp.concatenate([dst, jnp.zeros((npad,), jnp.int32)])
    pos = jnp.arange(_E_PAD, dtype=jnp.int32)
    real = srcp != dstp
    dummy = jnp.int32(_N) + pos % 16
    src_rm3 = jnp.where(real, srcp, dummy).reshape(_NTILES, _CPT, _CHUNK)

    bit = jnp.where(real, (dstp >= _HALF).astype(jnp.int32), pos % 2)
    c1 = jnp.cumsum(bit)
    ppos = jnp.where(bit == 1, c1 - 1, pos - c1)
    dstl = jnp.where(real, jnp.where(dstp < _HALF, dstp, dstp - _HALF),
                     jnp.int32(_HALF) + pos % 16)
    src_part = jnp.zeros((_NSC, _CAP), jnp.int32).at[bit, ppos].set(
        srcp, mode='drop')
    dst_part = jnp.full((_NSC, _CAP), _HALF, jnp.int32).at[bit, ppos].set(
        dstl, mode='drop')
    dst_part = dst_part.reshape(_NSC, _TPS, _CPT2, _CHUNK)
    soff = (src_part[None]
            + (jnp.arange(_NSLICE, dtype=jnp.int32) * _NPAD)[:, None, None]
            ).reshape(_NSLICE, _NSC, _TPS, _CPT2, _CHUNK)

    degs = _sc_degree(src_rm3)
    deg = (degs[0, :, 0] + degs[1, :, 0]).reshape(_NPAD, 1)

    xt = jnp.transpose(X, (0, 3, 1, 2))
    w1s = jnp.stack([jnp.transpose(w[:, :, 0, :], (2, 1, 0))
                     for w in (w11, w12, w13)])
    b1s = jnp.stack([b11, b12, b13])
    t0, u0 = _k1(xt, w1s, b1s, deg)

    acc1 = _sc_lap(u0.reshape(_NSLICE * _NPAD, _HID), soff, dst_part)
    u1, p = _k2(acc1, t0, deg, cheb_w[0], cheb_w[1])
    acc2 = _sc_lap(u1.reshape(_NSLICE * _NPAD, _HID), soff, dst_part)
    h = _k4(acc2, t0, p, deg, cheb_w[2], cheb_b.reshape(1, _HID))

    h5 = h.reshape(_B, _T1, _NPAD, _HID)
    w2s = jnp.stack([jnp.transpose(w[:, :, 0, :], (2, 1, 0))
                     for w in (w21, w22, w23)])
    b2s = jnp.stack([b21, b22, b23])
    o5 = _k5(h5, w2s, b2s, bn_w.reshape(_N, 1), bn_b.reshape(_N, 1))

    return jnp.transpose(o5, (2, 0, 1)).reshape(_SEQ, _B, _N, 1)


# speedup vs baseline: 1.8067x; 1.8067x over previous
import functools

import jax
import jax.numpy as jnp
from jax import lax
from jax.experimental import pallas as pl
from jax.experimental.pallas import tpu as pltpu
from jax.experimental.pallas import tpu_sc as plsc

_B = 2
_N = 10000
_CIN = 128
_HID = 128
_SEQ = 12
_T1 = 10
_T2 = 8
_NSLICE = _B * _T1
_OUT = 12

_NSC = 2
_TPS = 16
_NTILES = _NSC * _TPS
_CHUNK = 128
_CPT = 80
_EPT = _CHUNK * _CPT
_E_PAD = _NTILES * _EPT
_STRIPE = 632
_ROWS = _TPS * _STRIPE

_BN = 400
_NB = _N // _BN


def _dis_from_deg(deg_blk):
    return jnp.where(deg_blk > 0.0, lax.rsqrt(jnp.maximum(deg_blk, 1.0)), 0.0)


def _sc_degree(src_rm3):
    mesh = plsc.VectorSubcoreMesh(core_axis_name="c", subcore_axis_name="s",
                                  num_cores=_NSC, num_subcores=_TPS)

    @functools.partial(
        pl.kernel,
        out_type=jax.ShapeDtypeStruct((_NSC, _ROWS, _HID), jnp.float32),
        mesh=mesh,
        scratch_types=[
            pltpu.VMEM((_CPT, _CHUNK), jnp.int32),
            pltpu.VMEM((_CHUNK, _HID), jnp.float32),
            pltpu.VMEM_SHARED((_ROWS, _HID), jnp.float32),
        ],
    )
    def deg_kernel(src_hbm, out_hbm, idx_v, buf, acc):
        sc = lax.axis_index("c")
        ts = lax.axis_index("s")
        w = sc * _TPS + ts
        pltpu.sync_copy(src_hbm.at[w], idx_v)

        def fill(val):
            def frow(i, carry):
                def fcol(j, inner):
                    buf[i, pl.ds(j * 16, 16)] = jnp.full((16,), val,
                                                         jnp.float32)
                    return inner
                return lax.fori_loop(0, _HID // 16, fcol, carry)
            lax.fori_loop(0, _CHUNK, frow, 0)

        fill(0.0)
        for r in range(4):
            pltpu.sync_copy(buf,
                            acc.at[pl.ds(ts * _STRIPE + r * _CHUNK, _CHUNK)])
        rem = _STRIPE - 4 * _CHUNK
        pltpu.sync_copy(buf.at[pl.ds(0, rem)],
                        acc.at[pl.ds(ts * _STRIPE + 4 * _CHUNK, rem)])
        fill(1.0)
        plsc.subcore_barrier()

        def add_chunk(c, carry):
            pltpu.sync_copy(buf, acc.at[idx_v.at[c]], add=True)
            return carry

        lax.fori_loop(0, _CPT, add_chunk, 0)
        plsc.subcore_barrier()
        pltpu.sync_copy(acc.at[pl.ds(ts * _STRIPE, _STRIPE)],
                        out_hbm.at[sc, pl.ds(ts * _STRIPE, _STRIPE)])

    return deg_kernel(src_rm3)


def _sc_lap(u_flat, srcoff, dst_rm3):
    mesh = plsc.VectorSubcoreMesh(core_axis_name="c", subcore_axis_name="s",
                                   num_cores=_NSC, num_subcores=_TPS)

    hcpt = _CPT // 2

    @functools.partial(
        pl.kernel,
        out_type=jax.ShapeDtypeStruct((_NSC, _NSLICE, _ROWS, _HID),
                                      jnp.float32),
        mesh=mesh,
        scratch_types=[
            pltpu.VMEM((hcpt, _CHUNK), jnp.int32),
            pltpu.VMEM((hcpt, _CHUNK), jnp.int32),
            pltpu.VMEM((_CHUNK, _HID), jnp.float32),
            pltpu.VMEM((_CHUNK, _HID), jnp.float32),
            pltpu.VMEM_SHARED((_ROWS, _HID), jnp.float32),
            pltpu.SemaphoreType.DMA,
            pltpu.SemaphoreType.DMA,
        ],
    )
    def lap_kernel(u_hbm, src_hbm, dst_hbm, out_hbm,
                   src_v, dst_v, g0, g1, acc, sem0, sem1):
        sc = lax.axis_index("c")
        ts = lax.axis_index("s")
        w = sc * _TPS + ts
        gbufs = (g0, g1)
        sems = (sem0, sem1)

        def zero_stripe():
            def zrow(i, carry):
                def zcol(j, inner):
                    g0[i, pl.ds(j * 16, 16)] = jnp.zeros((16,), jnp.float32)
                    return inner
                return lax.fori_loop(0, _HID // 16, zcol, carry)

            lax.fori_loop(0, _CHUNK, zrow, 0)
            for r in range(4):
                pltpu.sync_copy(
                    g0, acc.at[pl.ds(ts * _STRIPE + r * _CHUNK, _CHUNK)])
            rem = _STRIPE - 4 * _CHUNK
            pltpu.sync_copy(g0.at[pl.ds(0, rem)],
                            acc.at[pl.ds(ts * _STRIPE + 4 * _CHUNK, rem)])

        zero_stripe()

        def half_body(s, h):
            pltpu.sync_copy(src_hbm.at[s, w, pl.ds(h * hcpt, hcpt)], src_v)
            pltpu.sync_copy(dst_hbm.at[w, pl.ds(h * hcpt, hcpt)], dst_v)
            for b in range(2):
                pltpu.async_copy(u_hbm.at[src_v.at[b]], gbufs[b], sems[b])

            def pair(p, inner):
                for b in range(2):
                    c = 2 * p + b
                    pltpu.make_async_copy(
                        u_hbm.at[pl.ds(0, _CHUNK)], gbufs[b],
                        sems[b]).wait()
                    pltpu.sync_copy(gbufs[b], acc.at[dst_v.at[c]], add=True)

                    @pl.when(p < hcpt // 2 - 1)
                    def _():
                        pltpu.async_copy(u_hbm.at[src_v.at[c + 2]],
                                         gbufs[b], sems[b])
                return inner

            lax.fori_loop(0, hcpt // 2, pair, 0)

        def slice_body(s, carry):
            plsc.subcore_barrier()
            half_body(s, 0)
            half_body(s, 1)
            plsc.subcore_barrier()
            pltpu.sync_copy(acc.at[pl.ds(ts * _STRIPE, _STRIPE)],
                            out_hbm.at[sc, s, pl.ds(ts * _STRIPE, _STRIPE)])
            zero_stripe()
            return carry

        lax.fori_loop(0, _NSLICE, slice_body, 0)

    return lap_kernel(u_flat, srcoff, dst_rm3)


def _k1_body(x0, x1, x2, w_ref, b_ref, deg_ref, t0_ref, u0_ref):
    xs = (x0[0, 0], x1[0, 0], x2[0, 0])
    gates = []
    for g in range(3):
        acc = jnp.zeros((_BN, _HID), jnp.float32)
        for k in range(3):
            acc = acc + jnp.dot(xs[k], w_ref[g, k],
                                preferred_element_type=jnp.float32)
        gates.append(acc + b_ref[g][None, :])
    p, q, r = gates
    h = jnp.maximum(p * jax.nn.sigmoid(q) + r, 0.0)
    dis = _dis_from_deg(deg_ref[...])
    t0_ref[0] = h
    u0_ref[0] = h * dis


def _k1(xt, w1s, b1s, deg):
    grid = (_NSLICE, _NB)

    def xmap(k):
        return lambda s, nb: (s // _T1, (s % _T1) + k, nb, 0)

    return pl.pallas_call(
        _k1_body,
        grid=grid,
        in_specs=[
            pl.BlockSpec((1, 1, _BN, _CIN), xmap(0)),
            pl.BlockSpec((1, 1, _BN, _CIN), xmap(1)),
            pl.BlockSpec((1, 1, _BN, _CIN), xmap(2)),
            pl.BlockSpec((3, 3, _CIN, _HID), lambda s, nb: (0, 0, 0, 0)),
            pl.BlockSpec((3, _HID), lambda s, nb: (0, 0)),
            pl.BlockSpec((_BN, 1), lambda s, nb: (nb, 0)),
        ],
        out_specs=[
            pl.BlockSpec((1, _BN, _HID), lambda s, nb: (s, nb, 0)),
            pl.BlockSpec((1, _BN, _HID), lambda s, nb: (s, nb, 0)),
        ],
        out_shape=[
            jax.ShapeDtypeStruct((_NSLICE, _N, _HID), jnp.float32),
            jax.ShapeDtypeStruct((_NSLICE, _N, _HID), jnp.float32),
        ],
    )(xt, xt, xt, w1s, b1s, deg)


def _k2_body(a0, a1, t0, deg_ref, w0_ref, w1_ref, u1_ref, p_ref):
    dis = _dis_from_deg(deg_ref[...])
    tx1 = -(dis * (a0[0, 0] + a1[0, 0]))
    u1_ref[0] = dis * tx1
    p_ref[0] = (jnp.dot(t0[0], w0_ref[...], preferred_element_type=jnp.float32)
                + jnp.dot(tx1, w1_ref[...], preferred_element_type=jnp.float32))


def _k2(acc1, t0, deg, w0, w1):
    grid = (_NSLICE, _NB)
    return pl.pallas_call(
        _k2_body,
        grid=grid,
        in_specs=[
            pl.BlockSpec((1, 1, _BN, _HID), lambda s, nb: (0, s, nb, 0)),
            pl.BlockSpec((1, 1, _BN, _HID), lambda s, nb: (1, s, nb, 0)),
            pl.BlockSpec((1, _BN, _HID), lambda s, nb: (s, nb, 0)),
            pl.BlockSpec((_BN, 1), lambda s, nb: (nb, 0)),
            pl.BlockSpec((_HID, _HID), lambda s, nb: (0, 0)),
            pl.BlockSpec((_HID, _HID), lambda s, nb: (0, 0)),
        ],
        out_specs=[
            pl.BlockSpec((1, _BN, _HID), lambda s, nb: (s, nb, 0)),
            pl.BlockSpec((1, _BN, _HID), lambda s, nb: (s, nb, 0)),
        ],
        out_shape=[
            jax.ShapeDtypeStruct((_NSLICE, _N, _HID), jnp.float32),
            jax.ShapeDtypeStruct((_NSLICE, _N, _HID), jnp.float32),
        ],
    )(acc1, acc1, t0, deg, w0, w1)


def _k4_body(a0, a1, t0, p, deg_ref, w2_ref, cb_ref, h_ref):
    dis = _dis_from_deg(deg_ref[...])
    tx2 = -2.0 * (dis * (a0[0, 0] + a1[0, 0])) - t0[0]
    out = (p[0]
           + jnp.dot(tx2, w2_ref[...], preferred_element_type=jnp.float32)
           + cb_ref[0][None, :])
    h_ref[0] = jnp.maximum(out, 0.0)


def _k4(acc2, t0, p, deg, w2, cb):
    grid = (_NSLICE, _NB)
    return pl.pallas_call(
        _k4_body,
        grid=grid,
        in_specs=[
            pl.BlockSpec((1, 1, _BN, _HID), lambda s, nb: (0, s, nb, 0)),
            pl.BlockSpec((1, 1, _BN, _HID), lambda s, nb: (1, s, nb, 0)),
            pl.BlockSpec((1, _BN, _HID), lambda s, nb: (s, nb, 0)),
            pl.BlockSpec((1, _BN, _HID), lambda s, nb: (s, nb, 0)),
            pl.BlockSpec((_BN, 1), lambda s, nb: (nb, 0)),
            pl.BlockSpec((_HID, _HID), lambda s, nb: (0, 0)),
            pl.BlockSpec((1, _HID), lambda s, nb: (0, 0)),
        ],
        out_specs=pl.BlockSpec((1, _BN, _HID), lambda s, nb: (s, nb, 0)),
        out_shape=jax.ShapeDtypeStruct((_NSLICE, _N, _HID), jnp.float32),
    )(acc2, acc2, t0, p, deg, w2, cb)


def _k5_body(h_ref, w_ref, b_ref, bnw_ref, bnb_ref, o_ref):
    outs = []
    for b in range(_B):
        for t in range(_T2):
            gates = []
            for g in range(3):
                acc = jnp.zeros((_BN, _OUT), jnp.float32)
                for k in range(3):
                    acc = acc + jnp.dot(h_ref[b, t + k], w_ref[g, k],
                                        preferred_element_type=jnp.float32)
                gates.append(acc + b_ref[g][None, :])
            p, q, r = gates
            outs.append(jnp.maximum(p * jax.nn.sigmoid(q) + r, 0.0))
    stack = jnp.stack(outs, axis=0)
    cnt = float(_B * _T2 * _OUT)
    mean = jnp.sum(jnp.sum(stack, axis=0), axis=1) / cnt
    msq = jnp.sum(jnp.sum(stack * stack, axis=0), axis=1) / cnt
    var = msq - mean * mean
    scale = lax.rsqrt(var + 1e-5) * bnw_ref[:, 0]
    shift = bnb_ref[:, 0]
    last = jnp.stack([outs[_T2 - 1], outs[2 * _T2 - 1]], axis=0)
    o_ref[...] = ((last - mean[None, :, None]) * scale[None, :, None]
                  + shift[None, :, None])


def _k5(h5, w2s, b2s, bnw, bnb):
    grid = (_NB,)
    return pl.pallas_call(
        _k5_body,
        grid=grid,
        in_specs=[
            pl.BlockSpec((_B, _T1, _BN, _HID), lambda nb: (0, 0, nb, 0)),
            pl.BlockSpec((3, 3, _HID, _OUT), lambda nb: (0, 0, 0, 0)),
            pl.BlockSpec((3, _OUT), lambda nb: (0, 0)),
            pl.BlockSpec((_BN, 1), lambda nb: (nb, 0)),
            pl.BlockSpec((_BN, 1), lambda nb: (nb, 0)),
        ],
        out_specs=pl.BlockSpec((_B, _BN, _OUT), lambda nb: (0, nb, 0)),
        out_shape=jax.ShapeDtypeStruct((_B, _N, _OUT), jnp.float32),
    )(h5, w2s, b2s, bnw, bnb)


def kernel(X, edge_index, w11, b11, w12, b12, w13, b13, cheb_w, cheb_b,
           w21, b21, w22, b22, w23, b23, bn_w, bn_b):
    src = edge_index[0].astype(jnp.int32)
    dst = edge_index[1].astype(jnp.int32)
    npad = _E_PAD - src.shape[0]
    srcp = jnp.concatenate([src, jnp.zeros((npad,), jnp.int32)])
    dstp = jnp.concatenate([dst, jnp.zeros((npad,), jnp.int32)])
    real = srcp != dstp
    dummy = jnp.int32(_N) + (jnp.arange(_E_PAD, dtype=jnp.int32) % 16)
    src_rm3 = jnp.where(real, srcp, dummy).reshape(_NTILES, _CPT, _CHUNK)
    dst_rm3 = jnp.where(real, dstp, dummy).reshape(_NTILES, _CPT, _CHUNK)
    soff = (srcp[None, :]
            + (jnp.arange(_NSLICE, dtype=jnp.int32) * _N)[:, None]
            ).reshape(_NSLICE, _NTILES, _CPT, _CHUNK)

    degs = _sc_degree(src_rm3)
    deg = (degs[0, :_N, 0] + degs[1, :_N, 0]).reshape(_N, 1)

    xt = jnp.transpose(X, (0, 3, 1, 2))
    w1s = jnp.stack([jnp.transpose(w[:, :, 0, :], (2, 1, 0))
                     for w in (w11, w12, w13)])
    b1s = jnp.stack([b11, b12, b13])
    t0, u0 = _k1(xt, w1s, b1s, deg)

    acc1 = _sc_lap(u0.reshape(_NSLICE * _N, _HID), soff, dst_rm3)
    u1, p = _k2(acc1, t0, deg, cheb_w[0], cheb_w[1])
    acc2 = _sc_lap(u1.reshape(_NSLICE * _N, _HID), soff, dst_rm3)
    h = _k4(acc2, t0, p, deg, cheb_w[2], cheb_b.reshape(1, _HID))

    h5 = h.reshape(_B, _T1, _N, _HID)
    w2s = jnp.stack([jnp.transpose(w[:, :, 0, :], (2, 1, 0))
                     for w in (w21, w22, w23)])
    b2s = jnp.stack([b21, b22, b23])
    o5 = _k5(h5, w2s, b2s, bn_w.reshape(_N, 1), bn_b.reshape(_N, 1))

    return jnp.transpose(o5, (2, 0, 1)).reshape(_SEQ, _B, _N, 1)
